# Initial kernel scaffold; baseline (speedup 1.0000x reference)
import functools
import math

import jax
import jax.numpy as jnp
from jax import lax
from jax.experimental import pallas as pl
from jax.experimental.pallas import tpu as pltpu
from jax.experimental.pallas import tpu_sc as plsc

EMB = 32
ROWS = 4096
COLS = 200
B_TOTAL = ROWS * COLS
NC, NS = 2, 16
NW = NC * NS
B_PER_W = B_TOTAL // NW
CHUNK = 1600
NCHUNK = B_PER_W // CHUNK
SCALE = math.sqrt(float(EMB))

_mesh = plsc.VectorSubcoreMesh(core_axis_name="c", subcore_axis_name="s")


@functools.partial(
    pl.kernel,
    mesh=_mesh,
    out_type=jax.ShapeDtypeStruct((B_TOTAL, EMB), jnp.float32),
    scratch_types=[
        pltpu.VMEM((CHUNK,), jnp.int32),
        pltpu.VMEM((CHUNK, EMB), jnp.float32),
        pltpu.SemaphoreType.DMA,
    ],
)
def _emb_lookup(tokens_hbm, table_hbm, out_hbm, idx_v, rows_v, sem):
    wid = lax.axis_index("s") * NC + lax.axis_index("c")
    base = wid * B_PER_W

    def chunk_body(g, carry):
        off = base + g * CHUNK
        pltpu.sync_copy(tokens_hbm.at[pl.ds(off, CHUNK)], idx_v)
        pltpu.async_copy(table_hbm.at[idx_v], rows_v, sem).wait()

        def scale_body(j, c):
            rows_v[j, 0:16] = rows_v[j, 0:16] * SCALE
            rows_v[j, 16:32] = rows_v[j, 16:32] * SCALE
            return c

        lax.fori_loop(0, CHUNK, scale_body, 0)
        pltpu.sync_copy(rows_v, out_hbm.at[pl.ds(off, CHUNK)])
        return carry

    lax.fori_loop(0, NCHUNK, chunk_body, 0)


def kernel(tokens, table):
    flat = tokens.reshape(B_TOTAL)
    out = _emb_lookup(flat, table)
    return out.reshape(ROWS, COLS, EMB)


# speedup vs baseline: 1.3079x; 1.3079x over previous
import functools
import math

import jax
import jax.numpy as jnp
from jax import lax
from jax.experimental import pallas as pl
from jax.experimental.pallas import tpu as pltpu
from jax.experimental.pallas import tpu_sc as plsc

EMB = 32
ROWS = 4096
COLS = 200
B_TOTAL = ROWS * COLS
NC, NS = 2, 16
NW = NC * NS
B_PER_W = B_TOTAL // NW
CHUNK = 1600
NCHUNK = B_PER_W // CHUNK
SCALE = math.sqrt(float(EMB))

_mesh = plsc.VectorSubcoreMesh(core_axis_name="c", subcore_axis_name="s")


@functools.partial(
    pl.kernel,
    mesh=_mesh,
    out_type=jax.ShapeDtypeStruct((B_TOTAL, EMB), jnp.float32),
    scratch_types=[
        pltpu.VMEM((CHUNK,), jnp.int32),
        pltpu.VMEM((CHUNK, EMB), jnp.float32),
        pltpu.SemaphoreType.DMA,
    ],
    compiler_params=pltpu.CompilerParams(use_tc_tiling_on_sc=False),
)
def _emb_lookup(tokens_hbm, table_hbm, out_hbm, idx_v, rows_v, sem):
    wid = lax.axis_index("s") * NC + lax.axis_index("c")
    base = wid * B_PER_W

    def chunk_body(g, carry):
        off = base + g * CHUNK
        pltpu.sync_copy(tokens_hbm.at[pl.ds(off, CHUNK)], idx_v)
        pltpu.async_copy(table_hbm.at[idx_v], rows_v, sem).wait()

        def scale_body(j, c):
            rows_v[j, 0:16] = rows_v[j, 0:16] * SCALE
            rows_v[j, 16:32] = rows_v[j, 16:32] * SCALE
            return c

        lax.fori_loop(0, CHUNK, scale_body, 0)
        pltpu.sync_copy(rows_v, out_hbm.at[pl.ds(off, CHUNK)])
        return carry

    lax.fori_loop(0, NCHUNK, chunk_body, 0)


def kernel(tokens, table):
    flat = tokens.reshape(B_TOTAL)
    out = _emb_lookup(flat, table)
    return out.reshape(ROWS, COLS, EMB)


# speedup vs baseline: 1.4720x; 1.1255x over previous
import functools
import math

import jax
import jax.numpy as jnp
from jax import lax
from jax.experimental import pallas as pl
from jax.experimental.pallas import tpu as pltpu
from jax.experimental.pallas import tpu_sc as plsc

EMB = 32
ROWS = 4096
COLS = 200
B_TOTAL = ROWS * COLS
NC, NS = 2, 16
NW = NC * NS
B_PER_W = B_TOTAL // NW
CHUNK = 1600
NCHUNK = B_PER_W // CHUNK
SCALE = math.sqrt(float(EMB))

_mesh = plsc.VectorSubcoreMesh(core_axis_name="c", subcore_axis_name="s")


@functools.partial(
    pl.kernel,
    mesh=_mesh,
    out_type=jax.ShapeDtypeStruct((B_TOTAL, EMB), jnp.float32),
    scratch_types=[
        pltpu.VMEM((CHUNK,), jnp.int32),
        pltpu.VMEM((CHUNK,), jnp.int32),
        pltpu.VMEM((CHUNK, EMB), jnp.float32),
        pltpu.VMEM((CHUNK, EMB), jnp.float32),
        pltpu.SemaphoreType.DMA,
        pltpu.SemaphoreType.DMA,
        pltpu.SemaphoreType.DMA,
        pltpu.SemaphoreType.DMA,
    ],
    compiler_params=pltpu.CompilerParams(use_tc_tiling_on_sc=False),
)
def _emb_lookup(tokens_hbm, table_hbm, out_hbm,
                idx0, idx1, rows0, rows1, g0, g1, o0, o1):
    wid = lax.axis_index("s") * NC + lax.axis_index("c")
    base = wid * B_PER_W
    idx = (idx0, idx1)
    rows = (rows0, rows1)
    gsem = (g0, g1)
    osem = (o0, o1)

    def _scale(rows_ref):
        @plsc.parallel_loop(0, CHUNK, step=1, unroll=8)
        def _(j):
            rows_ref[j, 0:16] = rows_ref[j, 0:16] * SCALE
            rows_ref[j, 16:32] = rows_ref[j, 16:32] * SCALE

    pltpu.sync_copy(tokens_hbm.at[pl.ds(base, CHUNK)], idx[0])
    gather = [pltpu.async_copy(table_hbm.at[idx[0]], rows[0], gsem[0]), None]
    out_dma = [None, None]

    for g in range(NCHUNK):
        b = g & 1
        b2 = b ^ 1
        if g + 1 < NCHUNK:
            if out_dma[b2] is not None:
                out_dma[b2].wait()
                out_dma[b2] = None
            off2 = base + (g + 1) * CHUNK
            pltpu.sync_copy(tokens_hbm.at[pl.ds(off2, CHUNK)], idx[b2])
            gather[b2] = pltpu.async_copy(table_hbm.at[idx[b2]], rows[b2], gsem[b2])
        gather[b].wait()
        _scale(rows[b])
        off = base + g * CHUNK
        out_dma[b] = pltpu.async_copy(rows[b], out_hbm.at[pl.ds(off, CHUNK)], osem[b])

    for d in out_dma:
        if d is not None:
            d.wait()


def kernel(tokens, table):
    flat = tokens.reshape(B_TOTAL)
    out = _emb_lookup(flat, table)
    return out.reshape(ROWS, COLS, EMB)
